# baseline (speedup 1.0000x reference)
import functools

import jax
import jax.numpy as jnp
from jax import lax
from jax.experimental import pallas as pl
from jax.experimental.pallas import tpu as pltpu
from jax.experimental.pallas import tpu_sc as plsc

G = 100000
B = 16384
D0, D1, D2 = 16, 64, 256
OUT_D = D0 + D1 + D2
DA = 128

_info = plsc.get_sparse_core_info()
NC, NS = _info.num_cores, _info.num_subcores
NW = NC * NS
BPW = B // NW
CH = 128
NCH = BPW // CH

_mesh = plsc.VectorSubcoreMesh(core_axis_name="c", subcore_axis_name="s")


@functools.partial(
    pl.kernel,
    mesh=_mesh,
    out_type=(
        jax.ShapeDtypeStruct((B, DA), jnp.float32),
        jax.ShapeDtypeStruct((B, D2), jnp.float32),
    ),
    compiler_params=pltpu.CompilerParams(use_tc_tiling_on_sc=True),
    scratch_types=[
        pltpu.VMEM((NCH, CH), jnp.int32),
        pltpu.VMEM((2 * CH, DA), jnp.float32),
        pltpu.VMEM((2 * CH, D2), jnp.float32),
        pltpu.SemaphoreType.DMA,
        pltpu.SemaphoreType.DMA,
    ],
)
def _sc_gather(x_hbm, taba_hbm, rep2_hbm, outa_hbm, out2_hbm,
               idx_v, rowsa_v, rows2_v, sema, sem2):
    wid = lax.axis_index("s") * NC + lax.axis_index("c")
    base = wid * BPW

    pltpu.sync_copy(x_hbm.at[pl.ds(wid * NCH, NCH)], idx_v)

    def fire2(j):
        return pltpu.async_copy(rep2_hbm.at[idx_v.at[j]],
                                rows2_v.at[pl.ds((j % 2) * CH, CH)], sem2)

    def firea(j):
        return pltpu.async_copy(taba_hbm.at[idx_v.at[j]],
                                rowsa_v.at[pl.ds((j % 2) * CH, CH)], sema)

    h2 = fire2(0)
    ha = firea(0)
    h2n = fire2(1)
    han = firea(1)
    for j in range(NCH):
        h2.wait()
        pltpu.sync_copy(rows2_v.at[pl.ds((j % 2) * CH, CH)],
                        out2_hbm.at[pl.ds(base + j * CH, CH)])
        h2 = h2n
        if j + 2 < NCH:
            h2n = fire2(j + 2)
        ha.wait()
        pltpu.sync_copy(rowsa_v.at[pl.ds((j % 2) * CH, CH)],
                        outa_hbm.at[pl.ds(base + j * CH, CH)])
        ha = han
        if j + 2 < NCH:
            han = firea(j + 2)


def kernel(x, rep0, rep1, rep2):
    x2 = x.astype(jnp.int32).reshape(B // CH, CH)
    r0 = rep0.reshape(G, D0)
    taba = jnp.concatenate([r0, rep1.reshape(G, D1), r0, r0, r0], axis=1)
    ga, g2 = _sc_gather(x2, taba, rep2.reshape(G, D2))
    return jnp.concatenate([ga[:, :D0 + D1], g2], axis=1)


# speedup vs baseline: 1.4054x; 1.4054x over previous
import functools

import jax
import jax.numpy as jnp
from jax import lax
from jax.experimental import pallas as pl
from jax.experimental.pallas import tpu as pltpu
from jax.experimental.pallas import tpu_sc as plsc

G = 100000
B = 16384
D0, D1, D2 = 16, 64, 256
OUT_D = D0 + D1 + D2

_info = plsc.get_sparse_core_info()
NC, NS = _info.num_cores, _info.num_subcores
NW = NC * NS
BPW = B // NW
CH = 128
NCH = BPW // CH

_mesh = plsc.VectorSubcoreMesh(core_axis_name="c", subcore_axis_name="s")


@functools.partial(
    pl.kernel,
    mesh=_mesh,
    out_type=(
        jax.ShapeDtypeStruct((B, D0), jnp.float32),
        jax.ShapeDtypeStruct((B, D1), jnp.float32),
    ),
    compiler_params=pltpu.CompilerParams(use_tc_tiling_on_sc=False),
    scratch_types=[
        pltpu.VMEM((NCH, CH), jnp.int32),
        pltpu.VMEM((BPW, D0), jnp.float32),
        pltpu.VMEM((BPW, D1), jnp.float32),
        pltpu.SemaphoreType.DMA,
        pltpu.SemaphoreType.DMA,
    ],
)
def _sc_gather_narrow(x_hbm, rep0_hbm, rep1_hbm, out0_hbm, out1_hbm,
                      idx_v, rows0_v, rows1_v, sem0, sem1):
    wid = lax.axis_index("s") * NC + lax.axis_index("c")
    base = wid * BPW

    pltpu.sync_copy(x_hbm.at[pl.ds(wid * NCH, NCH)], idx_v)

    h0 = [
        pltpu.async_copy(rep0_hbm.at[idx_v.at[j]],
                         rows0_v.at[pl.ds(j * CH, CH)], sem0)
        for j in range(NCH)
    ]
    h1 = [
        pltpu.async_copy(rep1_hbm.at[idx_v.at[j]],
                         rows1_v.at[pl.ds(j * CH, CH)], sem1)
        for j in range(NCH)
    ]
    for h in h0:
        h.wait()
    pltpu.sync_copy(rows0_v, out0_hbm.at[pl.ds(base, BPW)])
    for h in h1:
        h.wait()
    pltpu.sync_copy(rows1_v, out1_hbm.at[pl.ds(base, BPW)])


@functools.partial(
    pl.kernel,
    mesh=_mesh,
    out_type=jax.ShapeDtypeStruct((B, D2), jnp.float32),
    compiler_params=pltpu.CompilerParams(use_tc_tiling_on_sc=True),
    scratch_types=[
        pltpu.VMEM((NCH, CH), jnp.int32),
        pltpu.VMEM((2 * CH, D2), jnp.float32),
        pltpu.SemaphoreType.DMA,
    ],
)
def _sc_gather_wide(x_hbm, rep2_hbm, out2_hbm, idx_v, rows2_v, sem2):
    wid = lax.axis_index("s") * NC + lax.axis_index("c")
    base = wid * BPW

    pltpu.sync_copy(x_hbm.at[pl.ds(wid * NCH, NCH)], idx_v)

    def fire2(j):
        return pltpu.async_copy(rep2_hbm.at[idx_v.at[j]],
                                rows2_v.at[pl.ds((j % 2) * CH, CH)], sem2)

    h2 = fire2(0)
    h2n = fire2(1)
    for j in range(NCH):
        h2.wait()
        pltpu.sync_copy(rows2_v.at[pl.ds((j % 2) * CH, CH)],
                        out2_hbm.at[pl.ds(base + j * CH, CH)])
        h2 = h2n
        if j + 2 < NCH:
            h2n = fire2(j + 2)


def kernel(x, rep0, rep1, rep2):
    x2 = x.astype(jnp.int32).reshape(B // CH, CH)
    g0, g1 = _sc_gather_narrow(x2, rep0.reshape(G, D0), rep1.reshape(G, D1))
    g2 = _sc_gather_wide(x2, rep2.reshape(G, D2))
    return jnp.concatenate([g0, g1, g2], axis=1)


# speedup vs baseline: 1.4686x; 1.0450x over previous
import functools

import jax
import jax.numpy as jnp
from jax import lax
from jax.experimental import pallas as pl
from jax.experimental.pallas import tpu as pltpu
from jax.experimental.pallas import tpu_sc as plsc

G = 100000
B = 16384
D0, D1, D2 = 16, 64, 256
OUT_D = D0 + D1 + D2
DP = 128

_info = plsc.get_sparse_core_info()
NC, NS = _info.num_cores, _info.num_subcores
NW = NC * NS
BPW = B // NW
CH = 128
NCH = BPW // CH

_mesh = plsc.VectorSubcoreMesh(core_axis_name="c", subcore_axis_name="s")


@functools.partial(
    pl.kernel,
    mesh=_mesh,
    out_type=(
        jax.ShapeDtypeStruct((B, DP), jnp.float32),
        jax.ShapeDtypeStruct((B, DP), jnp.float32),
        jax.ShapeDtypeStruct((B, D2), jnp.float32),
    ),
    compiler_params=pltpu.CompilerParams(use_tc_tiling_on_sc=True),
    scratch_types=[
        pltpu.VMEM((NCH, CH), jnp.int32),
        pltpu.VMEM((CH, DP), jnp.float32),
        pltpu.VMEM((CH, DP), jnp.float32),
        pltpu.VMEM((2 * CH, D2), jnp.float32),
        pltpu.SemaphoreType.DMA,
        pltpu.SemaphoreType.DMA,
        pltpu.SemaphoreType.DMA,
    ],
)
def _sc_gather(x_hbm, rep0_hbm, rep1_hbm, rep2_hbm,
               out0_hbm, out1_hbm, out2_hbm,
               idx_v, rows0_v, rows1_v, rows2_v, sem0, sem1, sem2):
    wid = lax.axis_index("s") * NC + lax.axis_index("c")
    base = wid * BPW

    pltpu.sync_copy(x_hbm.at[pl.ds(wid * NCH, NCH)], idx_v)

    def fire2(j):
        return pltpu.async_copy(rep2_hbm.at[idx_v.at[j]],
                                rows2_v.at[pl.ds((j % 2) * CH, CH)], sem2)

    def fire0(j):
        return pltpu.async_copy(rep0_hbm.at[idx_v.at[j]], rows0_v, sem0)

    def fire1(j):
        return pltpu.async_copy(rep1_hbm.at[idx_v.at[j]], rows1_v, sem1)

    h2 = fire2(0)
    h0 = fire0(0)
    h1 = fire1(0)
    h2n = fire2(1)
    for j in range(NCH):
        h2.wait()
        pltpu.sync_copy(rows2_v.at[pl.ds((j % 2) * CH, CH)],
                        out2_hbm.at[pl.ds(base + j * CH, CH)])
        h2 = h2n
        if j + 2 < NCH:
            h2n = fire2(j + 2)
        h0.wait()
        pltpu.sync_copy(rows0_v, out0_hbm.at[pl.ds(base + j * CH, CH)])
        if j + 1 < NCH:
            h0 = fire0(j + 1)
        h1.wait()
        pltpu.sync_copy(rows1_v, out1_hbm.at[pl.ds(base + j * CH, CH)])
        if j + 1 < NCH:
            h1 = fire1(j + 1)


def kernel(x, rep0, rep1, rep2):
    x2 = x.astype(jnp.int32).reshape(B // CH, CH)
    r0 = jnp.pad(rep0.reshape(G, D0), ((0, 0), (0, DP - D0)))
    r1 = jnp.pad(rep1.reshape(G, D1), ((0, 0), (0, DP - D1)))
    g0, g1, g2 = _sc_gather(x2, r0, r1, rep2.reshape(G, D2))
    return jnp.concatenate([g0[:, :D0], g1[:, :D1], g2], axis=1)
